# Initial kernel scaffold; baseline (speedup 1.0000x reference)
import functools

import jax
import jax.numpy as jnp
from jax import lax
from jax.experimental import pallas as pl
from jax.experimental.pallas import tpu as pltpu
from jax.experimental.pallas import tpu_sc as plsc

_NUM_EMB = 1000000
_D = 64
_M = 99999
_P = 2147483647
_A = (98765431, 12345701)
_B = (7654321, 2468101)

_NC, _NS, _L = 2, 16, 16
_NW = _NC * _NS
_N = 16384 * 50
_C = 128


def _srl(x, n):
    return lax.shift_right_logical(x, jnp.int32(n))


def _fold(v):
    return (v & _P) + _srl(v, 31)


def _hash16(x, a, b):
    a_hi, a_lo = a >> 16, a & 0xFFFF
    x_hi = _srl(x, 16)
    x_lo = x & 0xFFFF
    t0 = a_lo * x_lo
    t0m = _fold(t0)
    t0r = jnp.where(t0m >= _P, t0m - _P, t0m)
    t1 = a_hi * x_lo + a_lo * x_hi
    t1c = ((t1 & 0x7FFF) << 16) + _srl(t1, 15)
    t2 = (a_hi * x_hi) * 2
    s1 = t0r + t1c
    s1m = _fold(s1)
    s1r = jnp.where(s1m >= _P, s1m - _P, s1m)
    s2 = s1r + t2 + b
    s2m = _fold(s2)
    h = jnp.where(s2m >= _P, s2m - _P, s2m)
    return h % _M + 1


def _body(emb_hbm, pw_hbm, idx_hbm, out_hbm,
          idx_v, b0_v, b1_v, p_v, r0_v, r1_v, out_v, sem):
    wid = lax.axis_index("s") * _NC + lax.axis_index("c")
    n_per_w = _N // _NW
    base_w = wid * n_per_w

    @pl.loop(0, n_per_w // _C)
    def _chunk(g):
        base = base_w + g * _C
        pltpu.sync_copy(idx_hbm.at[pl.ds(base, _C)], idx_v)
        for j in range(_C // _L):
            x = idx_v[pl.ds(j * _L, _L)]
            b0_v[pl.ds(j * _L, _L)] = _hash16(x, _A[0], _B[0])
            b1_v[pl.ds(j * _L, _L)] = _hash16(x, _A[1], _B[1])
        cp = pltpu.async_copy(pw_hbm.at[idx_v], p_v, sem)
        c0 = pltpu.async_copy(emb_hbm.at[b0_v], r0_v, sem)
        c1 = pltpu.async_copy(emb_hbm.at[b1_v], r1_v, sem)
        cp.wait()
        c0.wait()
        c1.wait()

        @pl.loop(0, _C)
        def _tok(t):
            tt = jnp.broadcast_to(t, (_L,))
            p0 = plsc.load_gather(p_v, [tt, jnp.zeros((_L,), jnp.int32)])
            p1 = plsc.load_gather(p_v, [tt, jnp.ones((_L,), jnp.int32)])
            for j in range(_D // _L):
                r0j = r0_v[t, pl.ds(j * _L, _L)]
                r1j = r1_v[t, pl.ds(j * _L, _L)]
                out_v[t, pl.ds(j * _L, _L)] = p0 * r0j + p1 * r1j

        pltpu.sync_copy(out_v, out_hbm.at[pl.ds(base, _C)])


_mesh = plsc.VectorSubcoreMesh(
    core_axis_name="c", subcore_axis_name="s", num_cores=_NC, num_subcores=_NS)

_sc_call = pl.kernel(
    _body,
    out_type=jax.ShapeDtypeStruct((_N, _D), jnp.float32),
    mesh=_mesh,
    scratch_types=[
        pltpu.VMEM((_C,), jnp.int32),
        pltpu.VMEM((_C,), jnp.int32),
        pltpu.VMEM((_C,), jnp.int32),
        pltpu.VMEM((_C, 2), jnp.float32),
        pltpu.VMEM((_C, _D), jnp.float32),
        pltpu.VMEM((_C, _D), jnp.float32),
        pltpu.VMEM((_C, _D), jnp.float32),
        pltpu.SemaphoreType.DMA,
    ],
)


def kernel(shared_embeddings, importance_weights, indices):
    idx32 = indices.reshape(-1).astype(jnp.int32)
    out = _sc_call(shared_embeddings.astype(jnp.float32),
                   importance_weights.astype(jnp.float32), idx32)
    return out.reshape(indices.shape + (_D,))


# speedup vs baseline: 3.4627x; 3.4627x over previous
import functools

import jax
import jax.numpy as jnp
from jax import lax
from jax.experimental import pallas as pl
from jax.experimental.pallas import tpu as pltpu
from jax.experimental.pallas import tpu_sc as plsc

_NUM_EMB = 1000000
_D = 64
_M = 99999
_P = 2147483647
_A = (98765431, 12345701)
_B = (7654321, 2468101)

_NC, _NS, _L = 2, 16, 16
_NW = _NC * _NS
_N = 16384 * 50
_C = 128


def _srl(x, n):
    return lax.shift_right_logical(x, jnp.int32(n))


def _fold(v):
    return (v & _P) + _srl(v, 31)


def _hash16(x, a, b):
    a_hi, a_lo = a >> 16, a & 0xFFFF
    x_hi = _srl(x, 16)
    x_lo = x & 0xFFFF
    t0 = a_lo * x_lo
    t0m = _fold(t0)
    t0r = jnp.where(t0m >= _P, t0m - _P, t0m)
    t1 = a_hi * x_lo + a_lo * x_hi
    t1c = ((t1 & 0x7FFF) << 16) + _srl(t1, 15)
    t2 = (a_hi * x_hi) * 2
    s1 = t0r + t1c
    s1m = _fold(s1)
    s1r = jnp.where(s1m >= _P, s1m - _P, s1m)
    s2 = s1r + t2 + b
    s2m = _fold(s2)
    h = jnp.where(s2m >= _P, s2m - _P, s2m)
    return h % _M + 1


def _body(emb_hbm, pw_hbm, idx_hbm, out_hbm,
          idx_v, b0_v, b1_v, i0_v, i1_v, p0_v, p1_v, r0_v, r1_v, out_v, sem):
    wid = (lax.axis_index("s").astype(jnp.int32) * jnp.int32(_NC)
           + lax.axis_index("c").astype(jnp.int32))
    n_per_w = _N // _NW
    base_w = wid * jnp.int32(n_per_w)

    @pl.loop(jnp.int32(0), jnp.int32(n_per_w // _C))
    def _chunk(g):
        base = base_w + g.astype(jnp.int32) * jnp.int32(_C)
        pltpu.sync_copy(idx_hbm.at[pl.ds(base, _C)], idx_v)
        for j in range(_C // _L):
            x = idx_v[pl.ds(j * _L, _L)]
            b0_v[pl.ds(j * _L, _L)] = _hash16(x, _A[0], _B[0])
            b1_v[pl.ds(j * _L, _L)] = _hash16(x, _A[1], _B[1])
            i0_v[pl.ds(j * _L, _L)] = x * 2
            i1_v[pl.ds(j * _L, _L)] = x * 2 + 1
        cp0 = pltpu.async_copy(pw_hbm.at[i0_v], p0_v, sem)
        cp1 = pltpu.async_copy(pw_hbm.at[i1_v], p1_v, sem)
        c0 = pltpu.async_copy(emb_hbm.at[b0_v], r0_v, sem)
        c1 = pltpu.async_copy(emb_hbm.at[b1_v], r1_v, sem)
        cp0.wait()
        cp1.wait()
        c0.wait()
        c1.wait()

        @pl.loop(jnp.int32(0), jnp.int32(_C))
        def _tok(t):
            t = t.astype(jnp.int32)
            tt = jnp.broadcast_to(t, (_L,))
            p0 = plsc.load_gather(p0_v, [tt])
            p1 = plsc.load_gather(p1_v, [tt])
            for j in range(_D // _L):
                r0j = r0_v[t, pl.ds(j * _L, _L)]
                r1j = r1_v[t, pl.ds(j * _L, _L)]
                out_v[t, pl.ds(j * _L, _L)] = p0 * r0j + p1 * r1j

        pltpu.sync_copy(out_v, out_hbm.at[pl.ds(base, _C)])


_mesh = plsc.VectorSubcoreMesh(
    core_axis_name="c", subcore_axis_name="s", num_cores=_NC, num_subcores=_NS)

_sc_call = pl.kernel(
    _body,
    out_type=jax.ShapeDtypeStruct((_N, _D), jnp.float32),
    mesh=_mesh,
    scratch_types=[
        pltpu.VMEM((_C,), jnp.int32),
        pltpu.VMEM((_C,), jnp.int32),
        pltpu.VMEM((_C,), jnp.int32),
        pltpu.VMEM((_C,), jnp.int32),
        pltpu.VMEM((_C,), jnp.int32),
        pltpu.VMEM((_C,), jnp.float32),
        pltpu.VMEM((_C,), jnp.float32),
        pltpu.VMEM((_C, _D), jnp.float32),
        pltpu.VMEM((_C, _D), jnp.float32),
        pltpu.VMEM((_C, _D), jnp.float32),
        pltpu.SemaphoreType.DMA,
    ],
    compiler_params=pltpu.CompilerParams(
        needs_layout_passes=False, use_tc_tiling_on_sc=False),
)


def kernel(shared_embeddings, importance_weights, indices):
    idx32 = indices.reshape(-1).astype(jnp.int32)
    out = _sc_call(shared_embeddings.astype(jnp.float32),
                   importance_weights.astype(jnp.float32).reshape(-1), idx32)
    return out.reshape(indices.shape + (_D,))


# speedup vs baseline: 4.0650x; 1.1739x over previous
import functools

import jax
import jax.numpy as jnp
from jax import lax
from jax.experimental import pallas as pl
from jax.experimental.pallas import tpu as pltpu
from jax.experimental.pallas import tpu_sc as plsc

_NUM_EMB = 1000000
_D = 64
_M = 99999
_P = 2147483647
_A = (98765431, 12345701)
_B = (7654321, 2468101)

_NC, _NS, _L = 2, 16, 16
_NW = _NC * _NS
_N = 16384 * 50
_C = 128
_NPW = _N // _NW
_G = _NPW // _C


def _srl(x, n):
    return lax.shift_right_logical(x, jnp.int32(n))


def _fold(v):
    return (v & _P) + _srl(v, 31)


def _hash16(x, a, b):
    a_hi, a_lo = a >> 16, a & 0xFFFF
    x_hi = _srl(x, 16)
    x_lo = x & 0xFFFF
    t0 = a_lo * x_lo
    t0m = _fold(t0)
    t0r = jnp.where(t0m >= _P, t0m - _P, t0m)
    t1 = a_hi * x_lo + a_lo * x_hi
    t1c = ((t1 & 0x7FFF) << 16) + _srl(t1, 15)
    t2 = (a_hi * x_hi) * 2
    s1 = t0r + t1c
    s1m = _fold(s1)
    s1r = jnp.where(s1m >= _P, s1m - _P, s1m)
    s2 = s1r + t2 + b
    s2m = _fold(s2)
    h = jnp.where(s2m >= _P, s2m - _P, s2m)
    return h % _M + 1


def _body(emb_hbm, pw_hbm, idx_hbm, out_hbm,
          idx_all, b0_v, b1_v, i0_v, i1_v, p0_v, p1_v, r0_v, r1_v, out_v,
          gsem, osem):
    wid = (lax.axis_index("s").astype(jnp.int32) * jnp.int32(_NC)
           + lax.axis_index("c").astype(jnp.int32))
    base_w = wid * jnp.int32(_NPW)

    pltpu.sync_copy(idx_hbm.at[pl.ds(base_w, _NPW)], idx_all)

    def _issue(g, s):
        off = g * jnp.int32(_C)
        for j in range(_C // _L):
            x = idx_all[pl.ds(off + jnp.int32(j * _L), _L)]
            b0_v[s][pl.ds(j * _L, _L)] = _hash16(x, _A[0], _B[0])
            b1_v[s][pl.ds(j * _L, _L)] = _hash16(x, _A[1], _B[1])
            i0_v[s][pl.ds(j * _L, _L)] = x * 2
            i1_v[s][pl.ds(j * _L, _L)] = x * 2 + 1
        pltpu.async_copy(pw_hbm.at[i0_v[s]], p0_v[s], gsem[s])
        pltpu.async_copy(pw_hbm.at[i1_v[s]], p1_v[s], gsem[s])
        pltpu.async_copy(emb_hbm.at[b0_v[s]], r0_v[s], gsem[s])
        pltpu.async_copy(emb_hbm.at[b1_v[s]], r1_v[s], gsem[s])

    def _drain_gathers(s):
        pltpu.make_async_copy(pw_hbm.at[i0_v[s]], p0_v[s], gsem[s]).wait()
        pltpu.make_async_copy(pw_hbm.at[i1_v[s]], p1_v[s], gsem[s]).wait()
        pltpu.make_async_copy(emb_hbm.at[b0_v[s]], r0_v[s], gsem[s]).wait()
        pltpu.make_async_copy(emb_hbm.at[b1_v[s]], r1_v[s], gsem[s]).wait()

    def _combine(s):
        @pl.loop(jnp.int32(0), jnp.int32(_C), step=jnp.int32(4))
        def _tok(t0):
            t0 = t0.astype(jnp.int32)
            for dt in range(4):
                t = t0 + jnp.int32(dt)
                tt = jnp.broadcast_to(t, (_L,))
                p0 = plsc.load_gather(p0_v[s], [tt])
                p1 = plsc.load_gather(p1_v[s], [tt])
                for j in range(_D // _L):
                    r0j = r0_v[s][t, pl.ds(j * _L, _L)]
                    r1j = r1_v[s][t, pl.ds(j * _L, _L)]
                    out_v[s][t, pl.ds(j * _L, _L)] = p0 * r0j + p1 * r1j

    def _drain_store(g, s):
        pltpu.make_async_copy(
            out_v[s], out_hbm.at[pl.ds(base_w + g * jnp.int32(_C), _C)],
            osem[s]).wait()

    _issue(jnp.int32(0), 0)

    @pl.loop(jnp.int32(0), jnp.int32(_G), step=jnp.int32(2))
    def _ring(g0):
        g0 = g0.astype(jnp.int32)
        for s in range(2):
            g = g0 + jnp.int32(s)

            @pl.when(g + 1 < _G)
            def _():
                _issue(g + 1, 1 - s)

            _drain_gathers(s)

            @pl.when(g >= 2)
            def _():
                _drain_store(g - 2, s)

            _combine(s)
            pltpu.async_copy(
                out_v[s], out_hbm.at[pl.ds(base_w + g * jnp.int32(_C), _C)],
                osem[s])

    _drain_store(jnp.int32(_G - 2), 0)
    _drain_store(jnp.int32(_G - 1), 1)


_mesh = plsc.VectorSubcoreMesh(
    core_axis_name="c", subcore_axis_name="s", num_cores=_NC, num_subcores=_NS)

_sc_call = pl.kernel(
    _body,
    out_type=jax.ShapeDtypeStruct((_N, _D), jnp.float32),
    mesh=_mesh,
    scratch_types=[
        pltpu.VMEM((_NPW,), jnp.int32),
        [pltpu.VMEM((_C,), jnp.int32) for _ in range(2)],
        [pltpu.VMEM((_C,), jnp.int32) for _ in range(2)],
        [pltpu.VMEM((_C,), jnp.int32) for _ in range(2)],
        [pltpu.VMEM((_C,), jnp.int32) for _ in range(2)],
        [pltpu.VMEM((_C,), jnp.float32) for _ in range(2)],
        [pltpu.VMEM((_C,), jnp.float32) for _ in range(2)],
        [pltpu.VMEM((_C, _D), jnp.float32) for _ in range(2)],
        [pltpu.VMEM((_C, _D), jnp.float32) for _ in range(2)],
        [pltpu.VMEM((_C, _D), jnp.float32) for _ in range(2)],
        [pltpu.SemaphoreType.DMA for _ in range(2)],
        [pltpu.SemaphoreType.DMA for _ in range(2)],
    ],
    compiler_params=pltpu.CompilerParams(
        needs_layout_passes=False, use_tc_tiling_on_sc=False),
)


def kernel(shared_embeddings, importance_weights, indices):
    idx32 = indices.reshape(-1).astype(jnp.int32)
    out = _sc_call(shared_embeddings.astype(jnp.float32),
                   importance_weights.astype(jnp.float32).reshape(-1), idx32)
    return out.reshape(indices.shape + (_D,))
